# baseline (speedup 1.0000x reference)
import jax
import jax.numpy as jnp
from jax.experimental import pallas as pl
from jax.experimental.pallas import tpu as pltpu

_D = 1024
_T = 50
_BT = 128


def _body(start_ref, cond_ref, w_ref, b_ref, te_ref, g_ref, be_ref,
          out_ref, ce_ref):
    t = pl.program_id(1)

    @pl.when(t == 0)
    def _():
        ce_ref[...] = jax.lax.dot_general(
            cond_ref[...], w_ref[...],
            dimension_numbers=(((1,), (1,)), ((), ())),
            preferred_element_type=jnp.float32,
        ) + b_ref[...]

    row = start_ref[0] + t
    q = ce_ref[...] + te_ref[pl.ds(row, 1), :]
    mean = jnp.mean(q, axis=1, keepdims=True)
    qc = q - mean
    var = jnp.mean(qc * qc, axis=1, keepdims=True)
    out_ref[...] = qc * jax.lax.rsqrt(var + 1e-5) * g_ref[...] + be_ref[...]


def kernel(T_future, batch_size, cond, time_embedding, W, b, gamma, beta):
    B = cond.shape[0]
    residual = (T_future - _T) + (batch_size - B)
    start = (1 + residual).astype(jnp.int32).reshape((1,))
    b2 = b.reshape((1, _D))
    g2 = gamma.reshape((1, _D))
    be2 = beta.reshape((1, _D))
    nb = B // _BT

    grid = (nb, _T)
    out = pl.pallas_call(
        _body,
        grid=grid,
        in_specs=[
            pl.BlockSpec(memory_space=pltpu.SMEM),
            pl.BlockSpec((_BT, cond.shape[1]), lambda i, t: (i, 0)),
            pl.BlockSpec((_D, cond.shape[1]), lambda i, t: (0, 0)),
            pl.BlockSpec((1, _D), lambda i, t: (0, 0)),
            pl.BlockSpec(time_embedding.shape, lambda i, t: (0, 0)),
            pl.BlockSpec((1, _D), lambda i, t: (0, 0)),
            pl.BlockSpec((1, _D), lambda i, t: (0, 0)),
        ],
        out_specs=pl.BlockSpec((_BT, _D), lambda i, t: (i, t)),
        out_shape=jax.ShapeDtypeStruct((B, _T * _D), jnp.float32),
        scratch_shapes=[pltpu.VMEM((_BT, _D), jnp.float32)],
    )(start, cond, W, b2, time_embedding, g2, be2)
    return out.reshape(B, _T, _D)


# speedup vs baseline: 1.3105x; 1.3105x over previous
import jax
import jax.numpy as jnp
from jax.experimental import pallas as pl
from jax.experimental.pallas import tpu as pltpu

_D = 1024
_T = 50
_BT = 128
_TC = 10


def _body(start_ref, cond_ref, w_ref, b_ref, te_ref, g_ref, be_ref,
          out_ref, ce_ref):
    tc = pl.program_id(1)

    @pl.when(tc == 0)
    def _():
        ce_ref[...] = jax.lax.dot_general(
            cond_ref[...], w_ref[...],
            dimension_numbers=(((1,), (1,)), ((), ())),
            preferred_element_type=jnp.float32,
        ) + b_ref[...]

    ce = ce_ref[...]
    g = g_ref[...]
    be = be_ref[...]
    row0 = start_ref[0] + tc * _TC
    for j in range(_TC):
        q = ce + te_ref[pl.ds(row0 + j, 1), :]
        mean = jnp.mean(q, axis=1, keepdims=True)
        qc = q - mean
        var = jnp.mean(qc * qc, axis=1, keepdims=True)
        out_ref[:, j * _D:(j + 1) * _D] = (
            qc * jax.lax.rsqrt(var + 1e-5) * g + be)


def kernel(T_future, batch_size, cond, time_embedding, W, b, gamma, beta):
    B = cond.shape[0]
    residual = (T_future - _T) + (batch_size - B)
    start = (1 + residual).astype(jnp.int32).reshape((1,))
    b2 = b.reshape((1, _D))
    g2 = gamma.reshape((1, _D))
    be2 = beta.reshape((1, _D))
    nb = B // _BT

    grid = (nb, _T // _TC)
    out = pl.pallas_call(
        _body,
        grid=grid,
        in_specs=[
            pl.BlockSpec(memory_space=pltpu.SMEM),
            pl.BlockSpec((_BT, cond.shape[1]), lambda i, t: (i, 0)),
            pl.BlockSpec((_D, cond.shape[1]), lambda i, t: (0, 0)),
            pl.BlockSpec((1, _D), lambda i, t: (0, 0)),
            pl.BlockSpec(time_embedding.shape, lambda i, t: (0, 0)),
            pl.BlockSpec((1, _D), lambda i, t: (0, 0)),
            pl.BlockSpec((1, _D), lambda i, t: (0, 0)),
        ],
        out_specs=pl.BlockSpec((_BT, _TC * _D), lambda i, t: (i, t)),
        out_shape=jax.ShapeDtypeStruct((B, _T * _D), jnp.float32),
        scratch_shapes=[pltpu.VMEM((_BT, _D), jnp.float32)],
    )(start, cond, W, b2, time_embedding, g2, be2)
    return out.reshape(B, _T, _D)


# speedup vs baseline: 1.7603x; 1.3432x over previous
import jax
import jax.numpy as jnp
from jax.experimental import pallas as pl
from jax.experimental.pallas import tpu as pltpu

_D = 1024
_T = 50
_BT = 64


def _body(start_ref, cond_ref, w_ref, b_ref, te_ref, g_ref, be_ref,
          out_ref, ce_ref):
    ce_ref[...] = jax.lax.dot_general(
        cond_ref[...], w_ref[...],
        dimension_numbers=(((1,), (1,)), ((), ())),
        preferred_element_type=jnp.float32,
    ) + b_ref[...]

    ce = ce_ref[...]
    g = g_ref[...]
    be = be_ref[...]
    row0 = start_ref[0]
    for t in range(_T):
        q = ce + te_ref[pl.ds(row0 + t, 1), :]
        mean = jnp.mean(q, axis=1, keepdims=True)
        qc = q - mean
        var = jnp.mean(qc * qc, axis=1, keepdims=True)
        out_ref[:, t, :] = qc * jax.lax.rsqrt(var + 1e-5) * g + be


def kernel(T_future, batch_size, cond, time_embedding, W, b, gamma, beta):
    B = cond.shape[0]
    residual = (T_future - _T) + (batch_size - B)
    start = (1 + residual).astype(jnp.int32).reshape((1,))
    b2 = b.reshape((1, _D))
    g2 = gamma.reshape((1, _D))
    be2 = beta.reshape((1, _D))
    nb = B // _BT

    out = pl.pallas_call(
        _body,
        grid=(nb,),
        in_specs=[
            pl.BlockSpec(memory_space=pltpu.SMEM),
            pl.BlockSpec((_BT, cond.shape[1]), lambda i: (i, 0)),
            pl.BlockSpec((_D, cond.shape[1]), lambda i: (0, 0)),
            pl.BlockSpec((1, _D), lambda i: (0, 0)),
            pl.BlockSpec(time_embedding.shape, lambda i: (0, 0)),
            pl.BlockSpec((1, _D), lambda i: (0, 0)),
            pl.BlockSpec((1, _D), lambda i: (0, 0)),
        ],
        out_specs=pl.BlockSpec((_BT, _T, _D), lambda i: (i, 0, 0)),
        out_shape=jax.ShapeDtypeStruct((B, _T, _D), jnp.float32),
        scratch_shapes=[pltpu.VMEM((_BT, _D), jnp.float32)],
    )(start, cond, W, b2, time_embedding, g2, be2)
    return out


# speedup vs baseline: 1.9455x; 1.1052x over previous
import jax
import jax.numpy as jnp
from jax.experimental import pallas as pl
from jax.experimental.pallas import tpu as pltpu

_D = 1024
_T = 50
_BT = 32


def _body(cond_ref, w_ref, b_ref, te_ref, g_ref, be_ref, out_ref, ce_ref):
    ce_ref[...] = jax.lax.dot_general(
        cond_ref[...], w_ref[...],
        dimension_numbers=(((1,), (1,)), ((), ())),
        preferred_element_type=jnp.float32,
    ) + b_ref[...]

    q = ce_ref[...][:, None, :] + te_ref[...][None, :, :]
    mean = jnp.mean(q, axis=2, keepdims=True)
    qc = q - mean
    var = jnp.mean(qc * qc, axis=2, keepdims=True)
    out_ref[...] = (qc * jax.lax.rsqrt(var + 1e-5) * g_ref[...][None, :, :]
                    + be_ref[...][None, :, :])


def kernel(T_future, batch_size, cond, time_embedding, W, b, gamma, beta):
    B = cond.shape[0]
    residual = (T_future - _T) + (batch_size - B)
    start = (1 + residual).astype(jnp.int32)
    te50 = jax.lax.dynamic_slice(time_embedding, (start, 0), (_T, _D))
    b2 = b.reshape((1, _D))
    g2 = gamma.reshape((1, _D))
    be2 = beta.reshape((1, _D))
    nb = B // _BT

    out = pl.pallas_call(
        _body,
        grid=(nb,),
        in_specs=[
            pl.BlockSpec((_BT, cond.shape[1]), lambda i: (i, 0)),
            pl.BlockSpec((_D, cond.shape[1]), lambda i: (0, 0)),
            pl.BlockSpec((1, _D), lambda i: (0, 0)),
            pl.BlockSpec((_T, _D), lambda i: (0, 0)),
            pl.BlockSpec((1, _D), lambda i: (0, 0)),
            pl.BlockSpec((1, _D), lambda i: (0, 0)),
        ],
        out_specs=pl.BlockSpec((_BT, _T, _D), lambda i: (i, 0, 0)),
        out_shape=jax.ShapeDtypeStruct((B, _T, _D), jnp.float32),
        scratch_shapes=[pltpu.VMEM((_BT, _D), jnp.float32)],
    )(cond, W, b2, te50, g2, be2)
    return out


# speedup vs baseline: 1.9464x; 1.0005x over previous
import jax
import jax.numpy as jnp
from jax.experimental import pallas as pl
from jax.experimental.pallas import tpu as pltpu

_D = 1024
_T = 50
_BT = 32


def _body(cond_ref, w_ref, b_ref, te_ref, g_ref, be_ref, out_ref, ce_ref):
    ce_ref[...] = jax.lax.dot_general(
        cond_ref[...], w_ref[...],
        dimension_numbers=(((1,), (1,)), ((), ())),
        preferred_element_type=jnp.float32,
    ) + b_ref[...]

    q = ce_ref[...][:, None, :] + te_ref[...][None, :, :]
    mean = jnp.mean(q, axis=2, keepdims=True)
    qc = q - mean
    var = jnp.mean(qc * qc, axis=2, keepdims=True)
    out_ref[...] = (qc * jax.lax.rsqrt(var + 1e-5) * g_ref[...][None, :, :]
                    + be_ref[...][None, :, :])


def kernel(T_future, batch_size, cond, time_embedding, W, b, gamma, beta):
    B = cond.shape[0]
    residual = (T_future - _T) + (batch_size - B)
    start = (1 + residual).astype(jnp.int32)
    te50 = jax.lax.dynamic_slice(time_embedding, (start, 0), (_T, _D))
    b2 = b.reshape((1, _D))
    g2 = gamma.reshape((1, _D))
    be2 = beta.reshape((1, _D))
    nb = B // _BT

    out = pl.pallas_call(
        _body,
        grid=(nb,),
        in_specs=[
            pl.BlockSpec((_BT, cond.shape[1]), lambda i: (i, 0)),
            pl.BlockSpec((_D, cond.shape[1]), lambda i: (0, 0)),
            pl.BlockSpec((1, _D), lambda i: (0, 0)),
            pl.BlockSpec((_T, _D), lambda i: (0, 0)),
            pl.BlockSpec((1, _D), lambda i: (0, 0)),
            pl.BlockSpec((1, _D), lambda i: (0, 0)),
        ],
        out_specs=pl.BlockSpec((_BT, _T, _D), lambda i: (i, 0, 0)),
        out_shape=jax.ShapeDtypeStruct((B, _T, _D), jnp.float32),
        scratch_shapes=[pltpu.VMEM((_BT, _D), jnp.float32)],
        compiler_params=pltpu.CompilerParams(
            dimension_semantics=("parallel",)),
    )(cond, W, b2, te50, g2, be2)
    return out


# speedup vs baseline: 5.1870x; 2.6650x over previous
import jax
import jax.numpy as jnp
from jax.experimental import pallas as pl
from jax.experimental.pallas import tpu as pltpu

_D = 1024
_T = 50
_BT = 32


def _body(cond_ref, w_ref, b_ref, te_ref, g_ref, be_ref, out_ref, ce_ref):
    ce_ref[...] = jax.lax.dot_general(
        cond_ref[...], w_ref[...],
        dimension_numbers=(((1,), (1,)), ((), ())),
        preferred_element_type=jnp.float32,
    ) + b_ref[...]

    q = ce_ref[...][None, :, :] + te_ref[...][:, None, :]
    mean = jnp.mean(q, axis=2, keepdims=True)
    qc = q - mean
    var = jnp.mean(qc * qc, axis=2, keepdims=True)
    out_ref[...] = (qc * jax.lax.rsqrt(var + 1e-5) * g_ref[...][None, :, :]
                    + be_ref[...][None, :, :])


def kernel(T_future, batch_size, cond, time_embedding, W, b, gamma, beta):
    B = cond.shape[0]
    residual = (T_future - _T) + (batch_size - B)
    start = (1 + residual).astype(jnp.int32)
    te50 = jax.lax.dynamic_slice(time_embedding, (start, 0), (_T, _D))
    b2 = b.reshape((1, _D))
    g2 = gamma.reshape((1, _D))
    be2 = beta.reshape((1, _D))
    nb = B // _BT

    out = pl.pallas_call(
        _body,
        grid=(nb,),
        in_specs=[
            pl.BlockSpec((_BT, cond.shape[1]), lambda i: (i, 0)),
            pl.BlockSpec((_D, cond.shape[1]), lambda i: (0, 0)),
            pl.BlockSpec((1, _D), lambda i: (0, 0)),
            pl.BlockSpec((_T, _D), lambda i: (0, 0)),
            pl.BlockSpec((1, _D), lambda i: (0, 0)),
            pl.BlockSpec((1, _D), lambda i: (0, 0)),
        ],
        out_specs=pl.BlockSpec((_T, _BT, _D), lambda i: (0, i, 0)),
        out_shape=jax.ShapeDtypeStruct((_T, B, _D), jnp.float32),
        scratch_shapes=[pltpu.VMEM((_BT, _D), jnp.float32)],
        compiler_params=pltpu.CompilerParams(
            dimension_semantics=("parallel",)),
    )(cond, W, b2, te50, g2, be2)
    return jnp.transpose(out, (1, 0, 2))


# speedup vs baseline: 5.7329x; 1.1052x over previous
import jax
import jax.numpy as jnp
from jax.experimental import pallas as pl
from jax.experimental.pallas import tpu as pltpu

_D = 1024
_T = 50
_BT = 32
_BM = 256


def _proj_body(cond_ref, w_ref, b_ref, ce_ref):
    ce_ref[...] = jax.lax.dot_general(
        cond_ref[...], w_ref[...],
        dimension_numbers=(((1,), (1,)), ((), ())),
        preferred_element_type=jnp.float32,
    ) + b_ref[...]


def _ln_body(ce_ref, te_ref, g_ref, be_ref, out_ref):
    q = ce_ref[...][None, :, :] + te_ref[...][:, None, :]
    mean = jnp.mean(q, axis=2, keepdims=True)
    qc = q - mean
    var = jnp.mean(qc * qc, axis=2, keepdims=True)
    out_ref[...] = (qc * jax.lax.rsqrt(var + 1e-5) * g_ref[...][None, :, :]
                    + be_ref[...][None, :, :])


def kernel(T_future, batch_size, cond, time_embedding, W, b, gamma, beta):
    B = cond.shape[0]
    residual = (T_future - _T) + (batch_size - B)
    start = (1 + residual).astype(jnp.int32)
    te50 = jax.lax.dynamic_slice(time_embedding, (start, 0), (_T, _D))
    b2 = b.reshape((1, _D))
    g2 = gamma.reshape((1, _D))
    be2 = beta.reshape((1, _D))

    ce = pl.pallas_call(
        _proj_body,
        grid=(B // _BM,),
        in_specs=[
            pl.BlockSpec((_BM, cond.shape[1]), lambda i: (i, 0)),
            pl.BlockSpec((_D, cond.shape[1]), lambda i: (0, 0)),
            pl.BlockSpec((1, _D), lambda i: (0, 0)),
        ],
        out_specs=pl.BlockSpec((_BM, _D), lambda i: (i, 0)),
        out_shape=jax.ShapeDtypeStruct((B, _D), jnp.float32),
        compiler_params=pltpu.CompilerParams(
            dimension_semantics=("parallel",)),
    )(cond, W, b2)

    out = pl.pallas_call(
        _ln_body,
        grid=(B // _BT,),
        in_specs=[
            pl.BlockSpec((_BT, _D), lambda i: (i, 0)),
            pl.BlockSpec((_T, _D), lambda i: (0, 0)),
            pl.BlockSpec((1, _D), lambda i: (0, 0)),
            pl.BlockSpec((1, _D), lambda i: (0, 0)),
        ],
        out_specs=pl.BlockSpec((_T, _BT, _D), lambda i: (0, i, 0)),
        out_shape=jax.ShapeDtypeStruct((_T, B, _D), jnp.float32),
        compiler_params=pltpu.CompilerParams(
            dimension_semantics=("parallel",)),
    )(ce, te50, g2, be2)
    return jnp.transpose(out, (1, 0, 2))
